# baseline (speedup 1.0000x reference)
import functools

import jax
import jax.numpy as jnp
from jax import lax
from jax.experimental import pallas as pl
from jax.experimental.pallas import tpu as pltpu
from jax.experimental.pallas import tpu_sc as plsc

VOC = 1_000_000
D = 32
B = 16384
L = 200

_info = plsc.get_sparse_core_info()
NC = _info.num_cores
NS = _info.num_subcores
NW = NC * NS

RW = B // NW
SCH_ROWS = 8
NSUP = RW // SCH_ROWS
HALF = L // 2
GPS = SCH_ROWS * 2
CHUNK = SCH_ROWS * L

_mesh = plsc.VectorSubcoreMesh(core_axis_name="c", subcore_axis_name="s")


TBLK = 2048
_PK_GRID = (VOC + TBLK - 1) // TBLK


def _pack_body(tt_ref, out_ref):
    t3 = tt_ref[...].T.reshape(TBLK // 4, 4, D)
    out_ref[...] = jnp.concatenate([t3[:, k, :] for k in range(4)], axis=1)


_pack = pl.pallas_call(
    _pack_body,
    grid=(_PK_GRID,),
    in_specs=[pl.BlockSpec((D, TBLK), lambda g: (0, g))],
    out_specs=pl.BlockSpec((TBLK // 4, 4 * D), lambda g: (g, 0)),
    out_shape=jax.ShapeDtypeStruct((VOC // 4, 4 * D), jnp.float32),
)


@functools.partial(
    pl.kernel,
    mesh=_mesh,
    compiler_params=pltpu.CompilerParams(use_tc_tiling_on_sc=False),
    out_type=jax.ShapeDtypeStruct((B, D), jnp.float32),
    scratch_types=[
        pltpu.VMEM((2, GPS, HALF), jnp.int32),
        pltpu.VMEM((2, CHUNK, D), jnp.float32),
        pltpu.VMEM((RW, D), jnp.float32),
        pltpu.SemaphoreType.DMA,
        pltpu.SemaphoreType.DMA,
    ],
)
def _emb(x2_hbm, table_hbm, out_hbm, idx_v, rows_v, out_v, sem0, sem1):
    wid = lax.axis_index("s") * NC + lax.axis_index("c")
    xbase = wid * (RW * 2)
    obase = wid * RW
    sems = (sem0, sem1)

    def load_and_fire(s, b):
        pltpu.sync_copy(
            x2_hbm.at[pl.ds(xbase + s * (SCH_ROWS * 2), SCH_ROWS * 2)],
            idx_v.at[b],
        )
        for g in range(GPS):
            pltpu.make_async_copy(
                table_hbm.at[idx_v.at[b, g]],
                rows_v.at[b, pl.ds(g * HALF, HALF)],
                sems[b],
            ).start()

    def drain(b):
        pltpu.make_async_copy(
            table_hbm.at[pl.ds(0, CHUNK)],
            rows_v.at[b],
            sems[b],
        ).wait()

    load_and_fire(0, 0)
    load_and_fire(1, 1)

    inv_l = jnp.float32(1.0 / L)

    def outer(i, carry):
        s0 = i * 2
        for b in range(2):
            s = s0 + b
            drain(b)
            for o in range(SCH_ROWS):
                def step(j, acc, _o=o, _b=b):
                    acc = list(acc)
                    for p in range(4):
                        r = _o * L + p * (L // 4) + j
                        acc[2 * p] = acc[2 * p] + rows_v[_b, r, pl.ds(0, 16)]
                        acc[2 * p + 1] = acc[2 * p + 1] + rows_v[_b, r, pl.ds(16, 16)]
                    return tuple(acc)

                z = jnp.zeros((16,), jnp.float32)
                acc = lax.fori_loop(0, L // 4, step, (z,) * 8, unroll=5)
                a0 = (acc[0] + acc[2]) + (acc[4] + acc[6])
                a1 = (acc[1] + acc[3]) + (acc[5] + acc[7])
                orow = s * SCH_ROWS + o
                out_v[orow, pl.ds(0, 16)] = a0 * inv_l
                out_v[orow, pl.ds(16, 16)] = a1 * inv_l

            @pl.when(s + 2 < NSUP)
            def _(s=s, b=b):
                load_and_fire(s + 2, b)
        return carry

    lax.fori_loop(0, NSUP // 2, outer, 0)
    pltpu.sync_copy(out_v, out_hbm.at[pl.ds(obase, RW)])


def kernel(x, word_pos, table):
    del word_pos
    x2 = x.reshape(2 * B, HALF).astype(jnp.int32)
    table_rm = _pack(table.T).reshape(VOC, D)
    return _emb(x2, table_rm)


# speedup vs baseline: 1.1080x; 1.1080x over previous
import functools

import jax
import jax.numpy as jnp
from jax import lax
from jax.experimental import pallas as pl
from jax.experimental.pallas import tpu as pltpu
from jax.experimental.pallas import tpu_sc as plsc

VOC = 1_000_000
D = 32
B = 16384
L = 200

_info = plsc.get_sparse_core_info()
NC = _info.num_cores
NS = _info.num_subcores
NW = NC * NS

RW = B // NW
SCH_ROWS = 8
NSUP = RW // SCH_ROWS
GLEN = 80
GPS = SCH_ROWS * L // GLEN
CHUNK = SCH_ROWS * L

_mesh = plsc.VectorSubcoreMesh(core_axis_name="c", subcore_axis_name="s")


TBLK = 2048
_PK_GRID = (VOC + TBLK - 1) // TBLK


def _pack_body(tt_ref, out_ref):
    t = tt_ref[...].T
    t4 = t.reshape(TBLK // 32, 4, 8, D)
    out_ref[...] = jnp.concatenate(
        [t4[:, k, :, :].reshape(TBLK // 4, D) for k in range(4)], axis=1
    )


_pack = pl.pallas_call(
    _pack_body,
    grid=(_PK_GRID,),
    in_specs=[pl.BlockSpec((D, TBLK), lambda g: (0, g))],
    out_specs=pl.BlockSpec((TBLK // 4, 4 * D), lambda g: (g, 0)),
    out_shape=jax.ShapeDtypeStruct((VOC // 4, 4 * D), jnp.float32),
)


@functools.partial(
    pl.kernel,
    mesh=_mesh,
    compiler_params=pltpu.CompilerParams(use_tc_tiling_on_sc=False),
    out_type=jax.ShapeDtypeStruct((B, D), jnp.float32),
    scratch_types=[
        pltpu.VMEM((2, GPS, GLEN), jnp.int32),
        pltpu.VMEM((2, CHUNK, D), jnp.float32),
        pltpu.VMEM((RW, D), jnp.float32),
        pltpu.SemaphoreType.DMA,
        pltpu.SemaphoreType.DMA,
    ],
)
def _emb(x3_hbm, table_hbm, out_hbm, idx_v, rows_v, out_v, sem0, sem1):
    wid = lax.axis_index("s") * NC + lax.axis_index("c")
    xbase = wid * (RW * L // GLEN)
    obase = wid * RW
    sems = (sem0, sem1)

    def load_and_fire(s, b):
        pltpu.sync_copy(
            x3_hbm.at[pl.ds(xbase + s * GPS, GPS)],
            idx_v.at[b],
        )
        for g in range(GPS):
            for c in range(GLEN // 16):
                v = idx_v[b, g, pl.ds(c * 16, 16)]
                idx_v[b, g, pl.ds(c * 16, 16)] = (
                    (v & -32) + ((v & 7) << 2) + ((v >> 3) & 3)
                )
        for g in range(GPS):
            pltpu.make_async_copy(
                table_hbm.at[idx_v.at[b, g]],
                rows_v.at[b, pl.ds(g * GLEN, GLEN)],
                sems[b],
            ).start()

    def drain(b):
        pltpu.make_async_copy(
            table_hbm.at[pl.ds(0, CHUNK)],
            rows_v.at[b],
            sems[b],
        ).wait()

    load_and_fire(0, 0)
    load_and_fire(1, 1)

    inv_l = jnp.float32(1.0 / L)

    def outer(i, carry):
        s0 = i * 2
        for b in range(2):
            s = s0 + b
            drain(b)
            for o in range(SCH_ROWS):
                def step(j, acc, _o=o, _b=b):
                    acc = list(acc)
                    for p in range(4):
                        r = _o * L + p * (L // 4) + j
                        acc[2 * p] = acc[2 * p] + rows_v[_b, r, pl.ds(0, 16)]
                        acc[2 * p + 1] = acc[2 * p + 1] + rows_v[_b, r, pl.ds(16, 16)]
                    return tuple(acc)

                z = jnp.zeros((16,), jnp.float32)
                acc = lax.fori_loop(0, L // 4, step, (z,) * 8, unroll=5)
                a0 = (acc[0] + acc[2]) + (acc[4] + acc[6])
                a1 = (acc[1] + acc[3]) + (acc[5] + acc[7])
                orow = s * SCH_ROWS + o
                out_v[orow, pl.ds(0, 16)] = a0 * inv_l
                out_v[orow, pl.ds(16, 16)] = a1 * inv_l

            @pl.when(s + 2 < NSUP)
            def _(s=s, b=b):
                load_and_fire(s + 2, b)
        return carry

    lax.fori_loop(0, NSUP // 2, outer, 0)
    pltpu.sync_copy(out_v, out_hbm.at[pl.ds(obase, RW)])


def kernel(x, word_pos, table):
    del word_pos
    x3 = x.reshape(B * L // GLEN, GLEN).astype(jnp.int32)
    table_rm = _pack(table.T).reshape(VOC, D)
    return _emb(x3, table_rm)


# speedup vs baseline: 1.4610x; 1.3187x over previous
import functools

import jax
import jax.numpy as jnp
from jax import lax
from jax.experimental import pallas as pl
from jax.experimental.pallas import tpu as pltpu
from jax.experimental.pallas import tpu_sc as plsc

VOC = 1_000_000
D = 32
B = 16384
L = 200

_info = plsc.get_sparse_core_info()
NC = _info.num_cores
NS = _info.num_subcores
NW = NC * NS

RW = B // NW
SCH_ROWS = 8
NSUP = RW // SCH_ROWS
GLEN = 80
GPS = SCH_ROWS * L // GLEN
CHUNK = SCH_ROWS * L

_mesh = plsc.VectorSubcoreMesh(core_axis_name="c", subcore_axis_name="s")


TBLK = 8192
QSH = (TBLK // 4).bit_length() - 1
_PK_GRID = (VOC + TBLK - 1) // TBLK


def _pack_body(tt_ref, out_ref):
    t = tt_ref[...].T
    t4 = t.reshape(TBLK // 32, 4, 8, D)
    out_ref[...] = jnp.concatenate(
        [t4[:, k, :, :].reshape(TBLK // 4, D) for k in range(4)], axis=1
    )


_pack = pl.pallas_call(
    _pack_body,
    grid=(_PK_GRID,),
    in_specs=[pl.BlockSpec((D, TBLK), lambda g: (0, g))],
    out_specs=pl.BlockSpec((TBLK // 4, 4 * D), lambda g: (g, 0)),
    out_shape=jax.ShapeDtypeStruct((VOC // 4, 4 * D), jnp.float32),
)


@functools.partial(
    pl.kernel,
    mesh=_mesh,
    compiler_params=pltpu.CompilerParams(use_tc_tiling_on_sc=False),
    out_type=jax.ShapeDtypeStruct((B, D), jnp.float32),
    scratch_types=[
        pltpu.VMEM((2, GPS, GLEN), jnp.int32),
        pltpu.VMEM((2, CHUNK, D), jnp.float32),
        pltpu.VMEM((RW, D), jnp.float32),
        pltpu.SemaphoreType.DMA,
        pltpu.SemaphoreType.DMA,
    ],
)
def _emb(x3_hbm, table_hbm, out_hbm, idx_v, rows_v, out_v, sem0, sem1):
    wid = lax.axis_index("s") * NC + lax.axis_index("c")
    xbase = wid * (RW * L // GLEN)
    obase = wid * RW
    sems = (sem0, sem1)

    def load_and_fire(s, b):
        pltpu.sync_copy(
            x3_hbm.at[pl.ds(xbase + s * GPS, GPS)],
            idx_v.at[b],
        )
        for g in range(GPS):
            for c in range(GLEN // 16):
                v = idx_v[b, g, pl.ds(c * 16, 16)]
                idx_v[b, g, pl.ds(c * 16, 16)] = (
                    (v & -32) + ((v & 7) << 2) + ((v >> 3) & 3)
                )
        for g in range(GPS):
            pltpu.make_async_copy(
                table_hbm.at[idx_v.at[b, g]],
                rows_v.at[b, pl.ds(g * GLEN, GLEN)],
                sems[b],
            ).start()

    def drain(b):
        pltpu.make_async_copy(
            table_hbm.at[pl.ds(0, CHUNK)],
            rows_v.at[b],
            sems[b],
        ).wait()

    load_and_fire(0, 0)
    load_and_fire(1, 1)

    inv_l = jnp.float32(1.0 / L)

    def outer(i, carry):
        s0 = i * 2
        for b in range(2):
            s = s0 + b
            drain(b)
            for o in range(SCH_ROWS):
                def step(j, acc, _o=o, _b=b):
                    acc = list(acc)
                    for p in range(4):
                        r = _o * L + p * (L // 4) + j
                        acc[2 * p] = acc[2 * p] + rows_v[_b, r, pl.ds(0, 16)]
                        acc[2 * p + 1] = acc[2 * p + 1] + rows_v[_b, r, pl.ds(16, 16)]
                    return tuple(acc)

                z = jnp.zeros((16,), jnp.float32)
                acc = lax.fori_loop(0, L // 4, step, (z,) * 8, unroll=5)
                a0 = (acc[0] + acc[2]) + (acc[4] + acc[6])
                a1 = (acc[1] + acc[3]) + (acc[5] + acc[7])
                orow = s * SCH_ROWS + o
                out_v[orow, pl.ds(0, 16)] = a0 * inv_l
                out_v[orow, pl.ds(16, 16)] = a1 * inv_l

            @pl.when(s + 2 < NSUP)
            def _(s=s, b=b):
                load_and_fire(s + 2, b)
        return carry

    lax.fori_loop(0, NSUP // 2, outer, 0)
    pltpu.sync_copy(out_v, out_hbm.at[pl.ds(obase, RW)])


def kernel(x, word_pos, table):
    del word_pos
    x3 = x.reshape(B * L // GLEN, GLEN).astype(jnp.int32)
    table_rm = _pack(table.T).reshape(VOC, D)
    return _emb(x3, table_rm)


# speedup vs baseline: 1.4685x; 1.0051x over previous
import functools

import jax
import jax.numpy as jnp
from jax import lax
from jax.experimental import pallas as pl
from jax.experimental.pallas import tpu as pltpu
from jax.experimental.pallas import tpu_sc as plsc

VOC = 1_000_000
D = 32
B = 16384
L = 200

_info = plsc.get_sparse_core_info()
NC = _info.num_cores
NS = _info.num_subcores
NW = NC * NS

RW = B // NW
SCH_ROWS = 8
NSUP = RW // SCH_ROWS
GLEN = 80
GPS = SCH_ROWS * L // GLEN
CHUNK = SCH_ROWS * L

_mesh = plsc.VectorSubcoreMesh(core_axis_name="c", subcore_axis_name="s")


TBLK = 8192
_PK_GRID = (VOC + TBLK - 1) // TBLK


def _pack_body(tt_ref, out_ref):
    t = tt_ref[...].T
    t4 = t.reshape(TBLK // 32, 4, 8, D)
    out_ref[...] = jnp.concatenate(
        [t4[:, k, :, :].reshape(TBLK // 4, D) for k in range(4)], axis=1
    )


_pack = pl.pallas_call(
    _pack_body,
    grid=(_PK_GRID,),
    in_specs=[pl.BlockSpec((D, TBLK), lambda g: (0, g))],
    out_specs=pl.BlockSpec((TBLK // 4, 4 * D), lambda g: (g, 0)),
    out_shape=jax.ShapeDtypeStruct((VOC // 4, 4 * D), jnp.float32),
)


@functools.partial(
    pl.kernel,
    mesh=_mesh,
    compiler_params=pltpu.CompilerParams(use_tc_tiling_on_sc=False),
    out_type=jax.ShapeDtypeStruct((B, D), jnp.float32),
    scratch_types=[
        pltpu.VMEM((2, GPS, GLEN), jnp.int32),
        pltpu.VMEM((2, CHUNK, D), jnp.float32),
        pltpu.VMEM((RW, D), jnp.float32),
        pltpu.SemaphoreType.DMA,
        pltpu.SemaphoreType.DMA,
    ],
)
def _emb(x3_hbm, table_hbm, out_hbm, idx_v, rows_v, out_v, sem0, sem1):
    wid = lax.axis_index("s") * NC + lax.axis_index("c")
    xbase = wid * (RW * L // GLEN)
    obase = wid * RW
    sems = (sem0, sem1)

    def load_and_fire(s, b):
        pltpu.sync_copy(
            x3_hbm.at[pl.ds(xbase + s * GPS, GPS)],
            idx_v.at[b],
        )
        for g in range(GPS):
            for c in range(GLEN // 16):
                v = idx_v[b, g, pl.ds(c * 16, 16)]
                idx_v[b, g, pl.ds(c * 16, 16)] = (
                    (v & -32) + ((v & 7) << 2) + ((v >> 3) & 3)
                )
        for g in range(GPS):
            pltpu.make_async_copy(
                table_hbm.at[idx_v.at[b, g]],
                rows_v.at[b, pl.ds(g * GLEN, GLEN)],
                sems[b],
            ).start()

    def drain(b):
        pltpu.make_async_copy(
            table_hbm.at[pl.ds(0, CHUNK)],
            rows_v.at[b],
            sems[b],
        ).wait()

    load_and_fire(0, 0)
    load_and_fire(1, 1)

    inv_l = jnp.float32(1.0 / L)

    def outer(i, carry):
        s0 = i * 2
        for b in range(2):
            s = s0 + b
            drain(b)
            for o in range(SCH_ROWS):
                def step(j, acc, _o=o, _b=b):
                    acc = list(acc)
                    for p in range(4):
                        r = _o * L + p * (L // 4) + j
                        acc[2 * p] = acc[2 * p] + rows_v[_b, r, pl.ds(0, 16)]
                        acc[2 * p + 1] = acc[2 * p + 1] + rows_v[_b, r, pl.ds(16, 16)]
                    return tuple(acc)

                z = jnp.zeros((16,), jnp.float32)
                acc = lax.fori_loop(0, L // 4, step, (z,) * 8, unroll=5)
                a0 = (acc[0] + acc[2]) + (acc[4] + acc[6])
                a1 = (acc[1] + acc[3]) + (acc[5] + acc[7])
                orow = s * SCH_ROWS + o
                out_v[orow, pl.ds(0, 16)] = a0 * inv_l
                out_v[orow, pl.ds(16, 16)] = a1 * inv_l

            @pl.when(s + 2 < NSUP)
            def _(s=s, b=b):
                load_and_fire(s + 2, b)
        return carry

    lax.fori_loop(0, NSUP // 2, outer, 0)
    pltpu.sync_copy(out_v, out_hbm.at[pl.ds(obase, RW)])


def kernel(x, word_pos, table):
    del word_pos
    x3 = x.reshape(B * L // GLEN, GLEN).astype(jnp.int32)
    table_rm = _pack(table.T).reshape(VOC, D)
    return _emb(x3, table_rm)


# speedup vs baseline: 1.6552x; 1.1272x over previous
import functools

import jax
import jax.numpy as jnp
from jax import lax
from jax.experimental import pallas as pl
from jax.experimental.pallas import tpu as pltpu
from jax.experimental.pallas import tpu_sc as plsc

VOC = 1_000_000
D = 32
B = 16384
L = 200

_info = plsc.get_sparse_core_info()
NC = _info.num_cores
NS = _info.num_subcores
NW = NC * NS

RW = B // NW
SCH_ROWS = 8
NSUP = RW // SCH_ROWS
GLEN = 80
GPS = SCH_ROWS * L // GLEN
CHUNK = SCH_ROWS * L

_mesh = plsc.VectorSubcoreMesh(core_axis_name="c", subcore_axis_name="s")


TBLK = 8192
QTR = TBLK // 4
_PK_GRID = (VOC + TBLK - 1) // TBLK
PK_ROWS = _PK_GRID * QTR


def _pack_body(tt_ref, out_ref):
    dlane = lax.broadcasted_iota(jnp.int32, (D, 4 * D), 0)
    clane = lax.broadcasted_iota(jnp.int32, (D, 4 * D), 1)
    acc = jnp.zeros((QTR, 4 * D), jnp.float32)
    for k in range(4):
        ek = (clane == D * k + dlane).astype(jnp.float32)
        acc = acc + lax.dot_general(
            tt_ref[:, k * QTR:(k + 1) * QTR],
            ek,
            (((0,), (0,)), ((), ())),
            preferred_element_type=jnp.float32,
        )
    out_ref[...] = acc


_pack = pl.pallas_call(
    _pack_body,
    grid=(_PK_GRID,),
    in_specs=[pl.BlockSpec((D, TBLK), lambda g: (0, g))],
    out_specs=pl.BlockSpec((QTR, 4 * D), lambda g: (g, 0)),
    out_shape=jax.ShapeDtypeStruct((PK_ROWS, 4 * D), jnp.float32),
)


@functools.partial(
    pl.kernel,
    mesh=_mesh,
    compiler_params=pltpu.CompilerParams(use_tc_tiling_on_sc=False),
    out_type=jax.ShapeDtypeStruct((B, D), jnp.float32),
    scratch_types=[
        pltpu.VMEM((2, GPS, GLEN), jnp.int32),
        pltpu.VMEM((2, CHUNK, D), jnp.float32),
        pltpu.VMEM((RW, D), jnp.float32),
        pltpu.SemaphoreType.DMA,
        pltpu.SemaphoreType.DMA,
    ],
)
def _emb(x3_hbm, table_hbm, out_hbm, idx_v, rows_v, out_v, sem0, sem1):
    wid = lax.axis_index("s") * NC + lax.axis_index("c")
    xbase = wid * (RW * L // GLEN)
    obase = wid * RW
    sems = (sem0, sem1)

    def load_and_fire(s, b):
        pltpu.sync_copy(
            x3_hbm.at[pl.ds(xbase + s * GPS, GPS)],
            idx_v.at[b],
        )
        for g in range(GPS):
            for c in range(GLEN // 16):
                v = idx_v[b, g, pl.ds(c * 16, 16)]
                idx_v[b, g, pl.ds(c * 16, 16)] = (
                    (v & -TBLK) + ((v & (QTR - 1)) << 2) + ((v >> 11) & 3)
                )
        for g in range(GPS):
            pltpu.make_async_copy(
                table_hbm.at[idx_v.at[b, g]],
                rows_v.at[b, pl.ds(g * GLEN, GLEN)],
                sems[b],
            ).start()

    def drain(b):
        pltpu.make_async_copy(
            table_hbm.at[pl.ds(0, CHUNK)],
            rows_v.at[b],
            sems[b],
        ).wait()

    load_and_fire(0, 0)
    load_and_fire(1, 1)

    inv_l = jnp.float32(1.0 / L)

    def outer(i, carry):
        s0 = i * 2
        for b in range(2):
            s = s0 + b
            drain(b)
            for o in range(SCH_ROWS):
                def step(j, acc, _o=o, _b=b):
                    acc = list(acc)
                    for p in range(4):
                        r = _o * L + p * (L // 4) + j
                        acc[2 * p] = acc[2 * p] + rows_v[_b, r, pl.ds(0, 16)]
                        acc[2 * p + 1] = acc[2 * p + 1] + rows_v[_b, r, pl.ds(16, 16)]
                    return tuple(acc)

                z = jnp.zeros((16,), jnp.float32)
                acc = lax.fori_loop(0, L // 4, step, (z,) * 8, unroll=5)
                a0 = (acc[0] + acc[2]) + (acc[4] + acc[6])
                a1 = (acc[1] + acc[3]) + (acc[5] + acc[7])
                orow = s * SCH_ROWS + o
                out_v[orow, pl.ds(0, 16)] = a0 * inv_l
                out_v[orow, pl.ds(16, 16)] = a1 * inv_l

            @pl.when(s + 2 < NSUP)
            def _(s=s, b=b):
                load_and_fire(s + 2, b)
        return carry

    lax.fori_loop(0, NSUP // 2, outer, 0)
    pltpu.sync_copy(out_v, out_hbm.at[pl.ds(obase, RW)])


def kernel(x, word_pos, table):
    del word_pos
    x3 = x.reshape(B * L // GLEN, GLEN).astype(jnp.int32)
    table_rm = _pack(table.T).reshape(PK_ROWS * 4, D)
    return _emb(x3, table_rm)
